# baseline (speedup 1.0000x reference)
import jax
import jax.numpy as jnp
from jax import lax
from jax.experimental import pallas as pl
from jax.experimental.pallas import tpu as pltpu
from jax.experimental.pallas import tpu_sc as plsc

_SEQ = 2048
_DIM = 1024
_NC = 2
_NS = 16
_NW = _NC * _NS
_ROWS = _SEQ // _NW


def _copy_body(pos_hbm, out_hbm):
    wid = lax.axis_index("s") * _NC + lax.axis_index("c")
    base = wid * _ROWS
    pltpu.sync_copy(pos_hbm.at[pl.ds(base, _ROWS)], out_hbm.at[pl.ds(base, _ROWS)])


def kernel(x, pos_embedding):
    mesh = plsc.VectorSubcoreMesh(core_axis_name="c", subcore_axis_name="s")
    out = pl.kernel(
        _copy_body,
        out_type=jax.ShapeDtypeStruct((_SEQ, _DIM), jnp.float32),
        mesh=mesh,
    )(pos_embedding)
    return out[None]


# speedup vs baseline: 10.4418x; 10.4418x over previous
import jax
import jax.numpy as jnp
from jax import lax
from jax.experimental import pallas as pl
from jax.experimental.pallas import tpu as pltpu
from jax.experimental.pallas import tpu_sc as plsc

_SEQ = 2048
_DIM = 1024
_NC = 2
_NS = 16
_NW = _NC * _NS
_ROWS = _SEQ // _NW


_NCHUNK = 4
_CH = _ROWS // _NCHUNK


def _copy_body(pos_hbm, out_hbm, buf0, buf1, rs0, rs1, ws0, ws1):
    wid = lax.axis_index("s") * _NC + lax.axis_index("c")
    base = wid * _ROWS
    bufs, rsems, wsems = (buf0, buf1), (rs0, rs1), (ws0, ws1)
    reads = [None] * _NCHUNK
    writes = [None] * _NCHUNK
    reads[0] = pltpu.async_copy(pos_hbm.at[pl.ds(base, _CH)], bufs[0], rsems[0])
    for i in range(_NCHUNK):
        cur = i % 2
        if i + 1 < _NCHUNK:
            if i - 1 >= 0:
                writes[i - 1].wait()
            reads[i + 1] = pltpu.async_copy(
                pos_hbm.at[pl.ds(base + (i + 1) * _CH, _CH)],
                bufs[1 - cur], rsems[1 - cur])
        reads[i].wait()
        writes[i] = pltpu.async_copy(
            bufs[cur], out_hbm.at[pl.ds(base + i * _CH, _CH)], wsems[cur])
    writes[_NCHUNK - 2].wait()
    writes[_NCHUNK - 1].wait()


def kernel(x, pos_embedding):
    mesh = plsc.VectorSubcoreMesh(core_axis_name="c", subcore_axis_name="s")
    out = pl.kernel(
        _copy_body,
        out_type=jax.ShapeDtypeStruct((_SEQ, _DIM), jnp.float32),
        scratch_types=[
            pltpu.VMEM((_CH, _DIM), jnp.float32),
            pltpu.VMEM((_CH, _DIM), jnp.float32),
            pltpu.SemaphoreType.DMA,
            pltpu.SemaphoreType.DMA,
            pltpu.SemaphoreType.DMA,
            pltpu.SemaphoreType.DMA,
        ],
        mesh=mesh,
    )(pos_embedding)
    return out[None]


# speedup vs baseline: 10.9750x; 1.0511x over previous
import jax
import jax.numpy as jnp
from jax import lax
from jax.experimental import pallas as pl
from jax.experimental.pallas import tpu as pltpu
from jax.experimental.pallas import tpu_sc as plsc

_SEQ = 2048
_DIM = 1024
_NC = 2
_NS = 16
_NW = _NC * _NS
_ROWS = _SEQ // _NW


def _copy_body(pos_hbm, out_hbm, buf):
    wid = lax.axis_index("s") * _NC + lax.axis_index("c")
    base = wid * _ROWS
    pltpu.sync_copy(pos_hbm.at[pl.ds(base, _ROWS)], buf)
    pltpu.sync_copy(buf, out_hbm.at[pl.ds(base, _ROWS)])


def kernel(x, pos_embedding):
    mesh = plsc.VectorSubcoreMesh(core_axis_name="c", subcore_axis_name="s")
    out = pl.kernel(
        _copy_body,
        out_type=jax.ShapeDtypeStruct((_SEQ, _DIM), jnp.float32),
        scratch_types=[
            pltpu.VMEM((_ROWS, _DIM), jnp.float32),
        ],
        mesh=mesh,
    )(pos_embedding)
    return out[None]
